# baseline (speedup 1.0000x reference)
import functools

import jax
import jax.numpy as jnp
from jax import lax
from jax.experimental import pallas as pl
from jax.experimental.pallas import tpu as pltpu
from jax.experimental.pallas import tpu_sc as plsc

VOCAB = 100000
D = 128
B = 1024
L = 200
LANES = 16
NC = 2
NS = 16
NW = NC * NS
SEQ_PER_W = B // NW
SCALE = float(D) ** 0.5
LA, LB = 104, 96
NBUF = 3


def _body(seq_hbm, table_hbm, pos_hbm, out_hbm,
          idx_all, pos_v, rows_v, sg0, sg1, sg2, sw0, sw1, sw2):
    sem_g = [sg0, sg1, sg2]
    sem_w = [sw0, sw1, sw2]
    wid = lax.axis_index("s") * NC + lax.axis_index("c")
    base = wid * SEQ_PER_W
    pltpu.sync_copy(pos_hbm, pos_v)
    pltpu.sync_copy(seq_hbm.at[pl.ds(base * L, SEQ_PER_W * L)], idx_all)

    def gather_pair(j, k):
        pltpu.async_copy(table_hbm.at[idx_all.at[pl.ds(j * L, LA)]],
                         rows_v.at[k, pl.ds(0, LA)], sem_g[k])
        pltpu.async_copy(table_hbm.at[idx_all.at[pl.ds(j * L + LA, LB)]],
                         rows_v.at[k, pl.ds(LA, LB)], sem_g[k])

    def gather_wait(j, k):
        pltpu.make_async_copy(table_hbm.at[idx_all.at[pl.ds(j * L, LA)]],
                              rows_v.at[k, pl.ds(0, LA)], sem_g[k]).wait()
        pltpu.make_async_copy(table_hbm.at[idx_all.at[pl.ds(j * L + LA, LB)]],
                              rows_v.at[k, pl.ds(LA, LB)], sem_g[k]).wait()

    def write_start(j, k):
        pltpu.async_copy(rows_v.at[k], out_hbm.at[base + j], sem_w[k])

    def write_wait(j, k):
        pltpu.make_async_copy(rows_v.at[k], out_hbm.at[base + j],
                              sem_w[k]).wait()

    def compute(k):
        def one_row(r, carry):
            for c in range(D // (2 * LANES)):
                w = pos_v[pl.ds(r * (D // 2) + c * LANES, LANES)]
                pa = lax.bitcast_convert_type(w << 16, jnp.float32)
                pb = lax.bitcast_convert_type(w & jnp.int32(-65536), jnp.float32)
                sl0 = pl.ds(c * 2 * LANES, LANES)
                sl1 = pl.ds(c * 2 * LANES + LANES, LANES)
                rows_v[k, r, sl0] = rows_v[k, r, sl0] * SCALE + pa
                rows_v[k, r, sl1] = rows_v[k, r, sl1] * SCALE + pb
            return carry

        lax.fori_loop(0, L, one_row, 0, unroll=False)

    def step(j, t, drain_write, prefetch):
        gather_wait(j, t)
        compute(t)
        write_start(j, t)
        kn = (t + 2) % NBUF
        if drain_write:
            write_wait(j - 1, kn)
        if prefetch:
            gather_pair(j + 2, kn)

    gather_pair(0, 0)
    gather_pair(1, 1)
    step(0, 0, drain_write=False, prefetch=True)
    step(1, 1, drain_write=True, prefetch=True)
    step(2, 2, drain_write=True, prefetch=True)

    def group(jj, carry):
        for t in range(NBUF):
            step(jj * NBUF + t, t, drain_write=True, prefetch=True)
        return carry

    lax.fori_loop(1, SEQ_PER_W // NBUF, group, 0, unroll=False)
    step(SEQ_PER_W - 2, 0, drain_write=True, prefetch=False)
    step(SEQ_PER_W - 1, 1, drain_write=True, prefetch=False)
    write_wait(SEQ_PER_W - 1, 1)


@jax.jit
def _pre_process(sequence, emb_table, pos_slice):
    f = functools.partial(
        pl.kernel,
        out_type=jax.ShapeDtypeStruct((B, L, D), jnp.float32),
        mesh=plsc.VectorSubcoreMesh(core_axis_name="c", subcore_axis_name="s"),
        scratch_types=[
            pltpu.VMEM((SEQ_PER_W * L,), jnp.int32),
            pltpu.VMEM((L * D // 2,), jnp.int32),
            pltpu.VMEM((NBUF, L, D), jnp.float32),
            pltpu.SemaphoreType.DMA,
            pltpu.SemaphoreType.DMA,
            pltpu.SemaphoreType.DMA,
            pltpu.SemaphoreType.DMA,
            pltpu.SemaphoreType.DMA,
            pltpu.SemaphoreType.DMA,
        ],
    )(_body)
    return f(sequence, emb_table, pos_slice)


def kernel(sequence, emb_table, pos_encoding, training=False, mask=None):
    seq = sequence.astype(jnp.int32).reshape(B * L)
    pos_slice = pos_encoding[0, :L, :].astype(jnp.float32)
    pos_bf = pos_slice.astype(jnp.bfloat16)
    bits = jax.lax.bitcast_convert_type(pos_bf, jnp.uint16).astype(jnp.uint32)
    bits = bits.reshape(L, D // 32, 2, 16)
    words = bits[:, :, 0, :] | (bits[:, :, 1, :] << 16)
    pos_packed = words.reshape(L * D // 2).astype(jnp.int32)
    return _pre_process(seq, emb_table, pos_packed)


# speedup vs baseline: 1.7166x; 1.7166x over previous
import functools

import jax
import jax.numpy as jnp
from jax import lax
from jax.experimental import pallas as pl
from jax.experimental.pallas import tpu as pltpu
from jax.experimental.pallas import tpu_sc as plsc

VOCAB = 100000
D = 128
B = 1024
L = 200
LANES = 16
NC = 2
NS = 16
NW = NC * NS
SEQ_PER_W = B // NW
SCALE = float(D) ** 0.5
LA, LB = 104, 96
NBUF = 6
NCHUNK = 2 * SEQ_PER_W
PREF = 4


def _body(seq_hbm, table_hbm, pos_hbm, out_hbm, idx_all, pos_v, rows_v,
          sg0, sg1, sg2, sg3, sg4, sg5, sw0, sw1, sw2, sw3, sw4, sw5):
    sem_g = [sg0, sg1, sg2, sg3, sg4, sg5]
    sem_w = [sw0, sw1, sw2, sw3, sw4, sw5]
    wid = lax.axis_index("s") * NC + lax.axis_index("c")
    base = wid * SEQ_PER_W
    pltpu.sync_copy(pos_hbm, pos_v)
    pltpu.sync_copy(seq_hbm.at[pl.ds(base * L, SEQ_PER_W * L)], idx_all)

    def chunk_refs(i, t, parity):
        ln = LA if parity == 0 else LB
        seq = i // 2
        idx = idx_all.at[pl.ds(seq * L + parity * LA, ln)]
        dst = rows_v.at[t, pl.ds(0, ln)]
        out = out_hbm.at[base + seq, pl.ds(parity * LA, ln)]
        return idx, dst, out

    def gather_start(i, t, parity):
        idx, dst, _ = chunk_refs(i, t, parity)
        pltpu.async_copy(table_hbm.at[idx], dst, sem_g[t])

    def gather_wait(i, t, parity):
        idx, dst, _ = chunk_refs(i, t, parity)
        pltpu.make_async_copy(table_hbm.at[idx], dst, sem_g[t]).wait()

    def write_start(i, t, parity):
        _, src, out = chunk_refs(i, t, parity)
        pltpu.async_copy(src, out, sem_w[t])

    def write_wait(i, t, parity):
        _, src, out = chunk_refs(i, t, parity)
        pltpu.make_async_copy(src, out, sem_w[t]).wait()

    def compute(t, parity):
        ln = LA if parity == 0 else LB

        def one_row(r, carry):
            pr = r + parity * LA
            for c in range(D // LANES):
                sl = pl.ds(c * LANES, LANES)
                rows_v[t, r, sl] = rows_v[t, r, sl] * SCALE + pos_v[pr, sl]
            return carry

        lax.fori_loop(0, ln, one_row, 0, unroll=False)

    def step(i, t, parity, drain_write, prefetch):
        kn = (t + PREF) % NBUF
        pn = parity
        if drain_write:
            write_wait(i - (NBUF - PREF), kn, pn)
        if prefetch:
            gather_start(i + PREF, kn, pn)
        gather_wait(i, t, parity)
        compute(t, parity)
        write_start(i, t, parity)

    for t in range(PREF):
        gather_start(t, t, t % 2)
    for i in range(NBUF):
        step(i, i, i % 2, drain_write=(i >= NBUF - PREF), prefetch=True)

    def group(g, carry):
        for t in range(NBUF):
            step(g * NBUF + t, t, t % 2, drain_write=True, prefetch=True)
        return carry

    lax.fori_loop(1, (NCHUNK - PREF) // NBUF, group, 0, unroll=False)
    for i in range(NCHUNK - PREF, NCHUNK):
        step(i, i % NBUF, i % 2, drain_write=True, prefetch=False)
    for i in range(NCHUNK - (NBUF - PREF), NCHUNK):
        write_wait(i, i % NBUF, i % 2)


@jax.jit
def _pre_process(sequence, emb_table, pos_slice):
    f = functools.partial(
        pl.kernel,
        out_type=jax.ShapeDtypeStruct((B, L, D), jnp.float32),
        mesh=plsc.VectorSubcoreMesh(core_axis_name="c", subcore_axis_name="s"),
        scratch_types=[
            pltpu.VMEM((SEQ_PER_W * L,), jnp.int32),
            pltpu.VMEM((L, D), jnp.float32),
            pltpu.VMEM((NBUF, LA, D), jnp.float32),
        ] + [pltpu.SemaphoreType.DMA] * (2 * NBUF),
    )(_body)
    return f(sequence, emb_table, pos_slice)


def kernel(sequence, emb_table, pos_encoding, training=False, mask=None):
    seq = sequence.astype(jnp.int32).reshape(B * L)
    pos_slice = pos_encoding[0, :L, :].astype(jnp.float32)
    return _pre_process(seq, emb_table, pos_slice)


# speedup vs baseline: 1.7312x; 1.0085x over previous
import functools

import jax
import jax.numpy as jnp
from jax import lax
from jax.experimental import pallas as pl
from jax.experimental.pallas import tpu as pltpu
from jax.experimental.pallas import tpu_sc as plsc

VOCAB = 100000
D = 128
B = 1024
L = 200
LANES = 16
NC = 2
NS = 16
NW = NC * NS
SEQ_PER_W = B // NW
SCALE = float(D) ** 0.5
LA, LB = 104, 96
NBUF = 6
NCHUNK = 2 * SEQ_PER_W
PREF = 4


def _body(seq_hbm, table_hbm, pos_hbm, out_hbm, idx_all, pos_v, rows_v,
          sg0, sg1, sg2, sg3, sg4, sg5, sw0, sw1, sw2, sw3, sw4, sw5):
    sem_g = [sg0, sg1, sg2, sg3, sg4, sg5]
    sem_w = [sw0, sw1, sw2, sw3, sw4, sw5]
    wid = lax.axis_index("s") * NC + lax.axis_index("c")
    base = wid * SEQ_PER_W
    pltpu.sync_copy(pos_hbm.at[0, pl.ds(0, L)], pos_v)
    pltpu.sync_copy(seq_hbm.at[pl.ds(base * L, SEQ_PER_W * L)], idx_all)

    def chunk_refs(i, t, parity):
        ln = LA if parity == 0 else LB
        seq = i // 2
        idx = idx_all.at[pl.ds(seq * L + parity * LA, ln)]
        dst = rows_v.at[t, pl.ds(0, ln)]
        out = out_hbm.at[base + seq, pl.ds(parity * LA, ln)]
        return idx, dst, out

    def gather_start(i, t, parity):
        idx, dst, _ = chunk_refs(i, t, parity)
        pltpu.async_copy(table_hbm.at[idx], dst, sem_g[t])

    def gather_wait(i, t, parity):
        idx, dst, _ = chunk_refs(i, t, parity)
        pltpu.make_async_copy(table_hbm.at[idx], dst, sem_g[t]).wait()

    def write_start(i, t, parity):
        _, src, out = chunk_refs(i, t, parity)
        pltpu.async_copy(src, out, sem_w[t])

    def write_wait(i, t, parity):
        _, src, out = chunk_refs(i, t, parity)
        pltpu.make_async_copy(src, out, sem_w[t]).wait()

    def compute(t, parity):
        ln = LA if parity == 0 else LB

        def one_row(r, carry):
            pr = r + parity * LA
            for c in range(D // LANES):
                sl = pl.ds(c * LANES, LANES)
                rows_v[t, r, sl] = rows_v[t, r, sl] * SCALE + pos_v[pr, sl]
            return carry

        lax.fori_loop(0, ln, one_row, 0, unroll=False)

    def step(i, t, parity, drain_write, prefetch):
        kn = (t + PREF) % NBUF
        pn = parity
        if drain_write:
            write_wait(i - (NBUF - PREF), kn, pn)
        if prefetch:
            gather_start(i + PREF, kn, pn)
        gather_wait(i, t, parity)
        compute(t, parity)
        write_start(i, t, parity)

    for t in range(PREF):
        gather_start(t, t, t % 2)
    for i in range(NBUF):
        step(i, i, i % 2, drain_write=(i >= NBUF - PREF), prefetch=True)

    def group(g, carry):
        for t in range(NBUF):
            step(g * NBUF + t, t, t % 2, drain_write=True, prefetch=True)
        return carry

    lax.fori_loop(1, (NCHUNK - PREF) // NBUF, group, 0, unroll=False)
    for i in range(NCHUNK - PREF, NCHUNK):
        step(i, i % NBUF, i % 2, drain_write=True, prefetch=False)
    for i in range(NCHUNK - (NBUF - PREF), NCHUNK):
        write_wait(i, i % NBUF, i % 2)


@jax.jit
def _pre_process(sequence, emb_table, pos_slice):
    f = functools.partial(
        pl.kernel,
        out_type=jax.ShapeDtypeStruct((B, L, D), jnp.float32),
        mesh=plsc.VectorSubcoreMesh(core_axis_name="c", subcore_axis_name="s"),
        scratch_types=[
            pltpu.VMEM((SEQ_PER_W * L,), jnp.int32),
            pltpu.VMEM((L, D), jnp.float32),
            pltpu.VMEM((NBUF, LA, D), jnp.float32),
        ] + [pltpu.SemaphoreType.DMA] * (2 * NBUF),
    )(_body)
    return f(sequence, emb_table, pos_slice)


def kernel(sequence, emb_table, pos_encoding, training=False, mask=None):
    seq = sequence.astype(jnp.int32).reshape(B * L)
    return _pre_process(seq, emb_table, pos_encoding)
